# baseline (speedup 1.0000x reference)
import jax
import jax.numpy as jnp
from jax import lax
from jax.experimental import pallas as pl
from jax.experimental.pallas import tpu as pltpu
from jax.experimental.pallas import tpu_sc as plsc

NUM_POINTS = 1000000
DIMS = 64
BATCH = 16384

NC = 2
NS = 16
NW = NC * NS
BPW = BATCH // NW
HALF = BPW // 2
GCH = 128
LANES = 16


def _sc_body(lorig_hbm, rorig_hbm, lrow_hbm, rrow_hbm, lhalf_hbm, rhalf_hbm,
             emb_hbm, blhs_hbm, brhs_hbm, out_hbm,
             lrowi_v, rrowi_v, lhalf_v, rhalf_v, lbidx_v, rbidx_v,
             lrows_v, rrows_v, lb_v, rb_v, m_v, out_v, sem, bsem):
    wid = lax.axis_index("s") * NC + lax.axis_index("c")
    base = wid * BPW

    pltpu.sync_copy(lrow_hbm.at[pl.ds(base, BPW)], lrowi_v)
    pltpu.sync_copy(rrow_hbm.at[pl.ds(base, BPW)], rrowi_v)
    pltpu.sync_copy(lhalf_hbm.at[pl.ds(base, BPW)], lhalf_v)
    pltpu.sync_copy(rhalf_hbm.at[pl.ds(base, BPW)], rhalf_v)
    pltpu.sync_copy(lorig_hbm.at[pl.ds(base, BPW)], lbidx_v)
    pltpu.sync_copy(rorig_hbm.at[pl.ds(base, BPW)], rbidx_v)

    bcopies = []
    for c in range(BPW // GCH):
        bcopies.append(pltpu.async_copy(
            blhs_hbm.at[lbidx_v.at[pl.ds(c * GCH, GCH)]],
            lb_v.at[pl.ds(c * GCH, GCH)], bsem))
        bcopies.append(pltpu.async_copy(
            brhs_hbm.at[rbidx_v.at[pl.ds(c * GCH, GCH)]],
            rb_v.at[pl.ds(c * GCH, GCH)], bsem))

    lane = lax.iota(jnp.int32, LANES)

    def half(h, carry):
        copies = []
        for c in range(HALF // GCH):
            copies.append(pltpu.async_copy(
                emb_hbm.at[lrowi_v.at[pl.ds(h * HALF + c * GCH, GCH)]],
                lrows_v.at[pl.ds(c * GCH, GCH)], sem))
            copies.append(pltpu.async_copy(
                emb_hbm.at[rrowi_v.at[pl.ds(h * HALF + c * GCH, GCH)]],
                rrows_v.at[pl.ds(c * GCH, GCH)], sem))
        for cp in copies:
            cp.wait()
        for blk in range(HALF // LANES):
            o = h * HALF + blk * LANES
            hlv = lhalf_v[pl.ds(o, LANES)]
            hrv = rhalf_v[pl.ds(o, LANES)]
            for j in range(LANES):
                p = blk * LANES + j
                sl = hlv[j]
                sr = hrv[j]
                acc = jnp.zeros((LANES,), jnp.float32)
                for k in range(DIMS // LANES):
                    lv = lrows_v[p, pl.ds(sl + k * LANES, LANES)]
                    rv = rrows_v[p, pl.ds(sr + k * LANES, LANES)]
                    d = lv - rv
                    acc = acc + d * d
                m_v[pl.ds(j * LANES, LANES)] = acc
            sqv = jnp.zeros((LANES,), jnp.float32)
            for k in range(LANES):
                sqv = sqv + plsc.load_gather(m_v, [lane * LANES + k])
            out_v[pl.ds(o, LANES)] = (
                lb_v[pl.ds(o, LANES)] + rb_v[pl.ds(o, LANES)] - sqv)
        return carry

    for bc in bcopies:
        bc.wait()
    lax.fori_loop(0, 2, half, 0)

    pltpu.sync_copy(out_v, out_hbm.at[pl.ds(base, BPW)])


@jax.jit
def _run(lorig, rorig, lrow, rrow, lhalf, rhalf, emb2, bias_lhs, bias_rhs):
    mesh = plsc.VectorSubcoreMesh(core_axis_name="c", subcore_axis_name="s")
    f = pl.kernel(
        _sc_body,
        out_type=jax.ShapeDtypeStruct((BATCH,), jnp.float32),
        mesh=mesh,
        compiler_params=pltpu.CompilerParams(needs_layout_passes=False),
        scratch_types=[
            pltpu.VMEM((BPW,), jnp.int32),
            pltpu.VMEM((BPW,), jnp.int32),
            pltpu.VMEM((BPW,), jnp.int32),
            pltpu.VMEM((BPW,), jnp.int32),
            pltpu.VMEM((BPW,), jnp.int32),
            pltpu.VMEM((BPW,), jnp.int32),
            pltpu.VMEM((HALF, 2 * DIMS), jnp.float32),
            pltpu.VMEM((HALF, 2 * DIMS), jnp.float32),
            pltpu.VMEM((BPW,), jnp.float32),
            pltpu.VMEM((BPW,), jnp.float32),
            pltpu.VMEM((LANES * LANES,), jnp.float32),
            pltpu.VMEM((BPW,), jnp.float32),
            pltpu.SemaphoreType.DMA,
            pltpu.SemaphoreType.DMA,
        ],
    )
    return f(lorig, rorig, lrow, rrow, lhalf, rhalf, emb2, bias_lhs, bias_rhs)


def kernel(input_triplet, embeddings, bias_lhs, bias_rhs):
    lorig = input_triplet[:, 0].astype(jnp.int32)
    rorig = input_triplet[:, -1].astype(jnp.int32)
    emb2 = embeddings.reshape(NUM_POINTS // 2, 2 * DIMS)
    return _run(lorig, rorig, lorig >> 1, rorig >> 1,
                (lorig & 1) * DIMS, (rorig & 1) * DIMS,
                emb2, bias_lhs, bias_rhs)
